# baseline (speedup 1.0000x reference)
import functools

import jax
import jax.numpy as jnp
from jax import lax
from jax.experimental import pallas as pl
from jax.experimental.pallas import tpu as pltpu
from jax.experimental.pallas import tpu_sc as plsc

_B, _L = 4096, 200
_N = _B * _L
_V = 1000000
_TOK_D = 64
_SMALL_D = 48
_OUT_D = _TOK_D + _SMALL_D
_PAD_D = 128
_NC, _NS = 2, 16
_NW = _NC * _NS
_ROWS_W = _N // _NW
_CG = 128
_NCH = _ROWS_W // _CG
_NBUF = 4
_BLK = 3200
_NBLK = _ROWS_W // _BLK
_VB = 512


def _detile_body(src_ref, dst_ref):
    x = src_ref[...]
    x3 = x.reshape(_TOK_D, _VB // 2, 2)
    y = x3.transpose(1, 0, 2)
    dst_ref[...] = y.transpose(0, 2, 1).reshape(_VB // 2, 128)


_detile = pl.pallas_call(
    _detile_body,
    grid=((_V + _VB - 1) // _VB,),
    in_specs=[pl.BlockSpec((_TOK_D, _VB), lambda j: (0, j))],
    out_specs=pl.BlockSpec((_VB // 2, 128), lambda j: (j, 0)),
    out_shape=jax.ShapeDtypeStruct((_V // 2, 128), jnp.float32),
)


def _sc_body(tok_hbm, wk_hbm, hr_hbm, du_hbm, tok_tab, comb_tab, out_hbm,
             tok_idx, cidx, wbuf, hbuf, dbuf, tok_rows, small_rows,
             sem_tok, sem_idx, sem_t, sem_s, sem_w):
    wid = lax.axis_index("c") * _NS + lax.axis_index("s")
    rbase = pl.multiple_of(wid * _ROWS_W, _ROWS_W)

    tok_cp = pltpu.async_copy(tok_hbm.at[pl.ds(rbase, _ROWS_W)], tok_idx,
                              sem_tok)
    for blk in range(_NBLK):
        off = rbase + blk * _BLK
        cw = pltpu.async_copy(wk_hbm.at[pl.ds(off, _BLK)], wbuf, sem_idx)
        ch = pltpu.async_copy(hr_hbm.at[pl.ds(off, _BLK)], hbuf, sem_idx)
        cd = pltpu.async_copy(du_hbm.at[pl.ds(off, _BLK)], dbuf, sem_idx)
        cw.wait()
        ch.wait()
        cd.wait()

        def fuse(j, _):
            s = pl.ds(pl.multiple_of(j * 16, 16), 16)
            w = wbuf[s]
            h = hbuf[s]
            d = dbuf[s]
            so = pl.ds(pl.multiple_of(blk * _BLK + j * 16, 16), 16)
            cidx[so] = (w * 24 + h) * 24 + d
            return _

        lax.fori_loop(0, _BLK // 16, fuse, 0)
    tok_cp.wait()

    def fire(g, slot):
        s = pl.ds(pl.multiple_of(g * _CG, _CG), _CG)
        pltpu.async_copy(tok_tab.at[tok_idx.at[s]], tok_rows.at[slot],
                         sem_t.at[slot])
        pltpu.async_copy(comb_tab.at[cidx.at[s]], small_rows.at[slot],
                         sem_s.at[slot])

    def drain(g, slot):
        s = pl.ds(pl.multiple_of(g * _CG, _CG), _CG)
        pltpu.make_async_copy(tok_tab.at[tok_idx.at[s]], tok_rows.at[slot],
                              sem_t.at[slot]).wait()
        pltpu.make_async_copy(comb_tab.at[cidx.at[s]], small_rows.at[slot],
                              sem_s.at[slot]).wait()

    def put(g, slot):
        r = pl.ds(pl.multiple_of(rbase + g * _CG, _CG), _CG)
        pltpu.async_copy(tok_rows.at[slot],
                         out_hbm.at[r, pl.ds(0, _TOK_D)], sem_w.at[slot])
        pltpu.async_copy(small_rows.at[slot],
                         out_hbm.at[r, pl.ds(_TOK_D, _SMALL_D)],
                         sem_w.at[slot])

    def wait_put(g, slot):
        r = pl.ds(pl.multiple_of(rbase + g * _CG, _CG), _CG)
        pltpu.make_async_copy(tok_rows.at[slot],
                              out_hbm.at[r, pl.ds(0, _TOK_D)],
                              sem_w.at[slot]).wait()
        pltpu.make_async_copy(small_rows.at[slot],
                              out_hbm.at[r, pl.ds(_TOK_D, _SMALL_D)],
                              sem_w.at[slot]).wait()

    for g in range(_NBUF - 1):
        fire(g, g)

    def step(i, _):
        for b in range(_NBUF):
            g = i * _NBUF + b
            b3 = (b + _NBUF - 1) % _NBUF

            @pl.when(g >= 1)
            def _wp():
                wait_put(g - 1, b3)

            @pl.when(g + _NBUF - 1 < _NCH)
            def _f():
                fire(g + _NBUF - 1, b3)

            drain(g, b)
            put(g, b)
        return _

    lax.fori_loop(0, _NCH // _NBUF, step, 0)
    wait_put(_NCH - 1, (_NCH - 1) % _NBUF)


_sc_call = pl.kernel(
    _sc_body,
    out_type=jax.ShapeDtypeStruct((_N, _PAD_D), jnp.float32),
    mesh=plsc.VectorSubcoreMesh(core_axis_name="c", subcore_axis_name="s"),
    compiler_params=pltpu.CompilerParams(use_tc_tiling_on_sc=False),
    scratch_types=[
        pltpu.VMEM((_ROWS_W,), jnp.int32),
        pltpu.VMEM((_ROWS_W,), jnp.int32),
        pltpu.VMEM((_BLK,), jnp.int32),
        pltpu.VMEM((_BLK,), jnp.int32),
        pltpu.VMEM((_BLK,), jnp.int32),
        pltpu.VMEM((_NBUF, _CG, _TOK_D), jnp.float32),
        pltpu.VMEM((_NBUF, _CG, _SMALL_D), jnp.float32),
        pltpu.SemaphoreType.DMA,
        pltpu.SemaphoreType.DMA,
        pltpu.SemaphoreType.DMA((_NBUF,)),
        pltpu.SemaphoreType.DMA((_NBUF,)),
        pltpu.SemaphoreType.DMA((_NBUF,)),
    ],
)


def _tout_body(src_ref, dst_ref):
    dst_ref[0] = src_ref[:, : _OUT_D].T


_tout = pl.pallas_call(
    _tout_body,
    grid=(_L,),
    in_specs=[pl.BlockSpec((_B, _PAD_D), lambda l: (l, 0))],
    out_specs=pl.BlockSpec((1, _OUT_D, _B), lambda l: (l, 0, 0)),
    out_shape=jax.ShapeDtypeStruct((_L, _OUT_D, _B), jnp.float32),
)


def kernel(token, week, hour, duration, token_table, week_table, hour_table,
           duration_table):
    comb = jnp.concatenate([
        jnp.broadcast_to(week_table[:, None, None, :], (7, 24, 24, 16)),
        jnp.broadcast_to(hour_table[None, :, None, :], (7, 24, 24, 16)),
        jnp.broadcast_to(duration_table[None, None, :, :], (7, 24, 24, 16)),
    ], axis=-1).reshape(7 * 24 * 24, _SMALL_D)

    tab_lin = _detile(token_table.T).reshape(_V, _TOK_D)
    rows = _sc_call(token.T.reshape(_N), week.T.reshape(_N),
                    hour.T.reshape(_N), duration.T.reshape(_N),
                    tab_lin, comb)
    out3 = _tout(rows)
    return out3.transpose(2, 0, 1)


# speedup vs baseline: 9.6657x; 9.6657x over previous
import functools

import jax
import jax.numpy as jnp
from jax import lax
from jax.experimental import pallas as pl
from jax.experimental.pallas import tpu as pltpu
from jax.experimental.pallas import tpu_sc as plsc

_B, _L = 4096, 200
_N = _B * _L
_V = 1000000
_TOK_D = 64
_SMALL_D = 48
_OUT_D = _TOK_D + _SMALL_D
_PAD_D = 128
_NC, _NS = 2, 16
_NW = _NC * _NS
_ROWS_W = _N // _NW
_CG = 128
_NCH = _ROWS_W // _CG
_NBUF = 4
_BLK = 3200
_NBLK = _ROWS_W // _BLK
_VB = 8192


def _detile_body(src_ref, dst_ref):
    x = src_ref[...]
    xp = jnp.pad(x, ((0, 128 - _TOK_D), (0, 0)))
    t = xp.T
    z = t.reshape(_VB // 2, 2, 128)
    dst_ref[:, 0:_TOK_D] = z[:, 0, 0:_TOK_D]
    dst_ref[:, _TOK_D:128] = z[:, 1, 0:_TOK_D]


_detile = pl.pallas_call(
    _detile_body,
    grid=((_V + _VB - 1) // _VB,),
    in_specs=[pl.BlockSpec((_TOK_D, _VB), lambda j: (0, j))],
    out_specs=pl.BlockSpec((_VB // 2, 128), lambda j: (j, 0)),
    out_shape=jax.ShapeDtypeStruct((_V // 2, 128), jnp.float32),
)


def _sc_body(tok_hbm, wk_hbm, hr_hbm, du_hbm, tok_tab, comb_tab, out_hbm,
             tok_idx, cidx, wbuf, hbuf, dbuf, tok_rows, small_rows,
             sem_tok, sem_idx, sem_t, sem_s, sem_w):
    wid = lax.axis_index("c") * _NS + lax.axis_index("s")
    rbase = pl.multiple_of(wid * _ROWS_W, _ROWS_W)

    tok_cp = pltpu.async_copy(tok_hbm.at[pl.ds(rbase, _ROWS_W)], tok_idx,
                              sem_tok)
    for blk in range(_NBLK):
        off = rbase + blk * _BLK
        cw = pltpu.async_copy(wk_hbm.at[pl.ds(off, _BLK)], wbuf, sem_idx)
        ch = pltpu.async_copy(hr_hbm.at[pl.ds(off, _BLK)], hbuf, sem_idx)
        cd = pltpu.async_copy(du_hbm.at[pl.ds(off, _BLK)], dbuf, sem_idx)
        cw.wait()
        ch.wait()
        cd.wait()

        def fuse(j, _):
            s = pl.ds(pl.multiple_of(j * 16, 16), 16)
            w = wbuf[s]
            h = hbuf[s]
            d = dbuf[s]
            so = pl.ds(pl.multiple_of(blk * _BLK + j * 16, 16), 16)
            cidx[so] = (w * 24 + h) * 24 + d
            return _

        lax.fori_loop(0, _BLK // 16, fuse, 0)
    tok_cp.wait()

    def fire(g, slot):
        s = pl.ds(pl.multiple_of(g * _CG, _CG), _CG)
        pltpu.async_copy(tok_tab.at[tok_idx.at[s]], tok_rows.at[slot],
                         sem_t.at[slot])
        pltpu.async_copy(comb_tab.at[cidx.at[s]], small_rows.at[slot],
                         sem_s.at[slot])

    def drain(g, slot):
        s = pl.ds(pl.multiple_of(g * _CG, _CG), _CG)
        pltpu.make_async_copy(tok_tab.at[tok_idx.at[s]], tok_rows.at[slot],
                              sem_t.at[slot]).wait()
        pltpu.make_async_copy(comb_tab.at[cidx.at[s]], small_rows.at[slot],
                              sem_s.at[slot]).wait()

    def put(g, slot):
        r = pl.ds(pl.multiple_of(rbase + g * _CG, _CG), _CG)
        pltpu.async_copy(tok_rows.at[slot],
                         out_hbm.at[r, pl.ds(0, _TOK_D)], sem_w.at[slot])
        pltpu.async_copy(small_rows.at[slot],
                         out_hbm.at[r, pl.ds(_TOK_D, _SMALL_D)],
                         sem_w.at[slot])

    def wait_put(g, slot):
        r = pl.ds(pl.multiple_of(rbase + g * _CG, _CG), _CG)
        pltpu.make_async_copy(tok_rows.at[slot],
                              out_hbm.at[r, pl.ds(0, _TOK_D)],
                              sem_w.at[slot]).wait()
        pltpu.make_async_copy(small_rows.at[slot],
                              out_hbm.at[r, pl.ds(_TOK_D, _SMALL_D)],
                              sem_w.at[slot]).wait()

    for g in range(_NBUF - 1):
        fire(g, g)

    def step(i, _):
        for b in range(_NBUF):
            g = i * _NBUF + b
            b3 = (b + _NBUF - 1) % _NBUF

            @pl.when(g >= 1)
            def _wp():
                wait_put(g - 1, b3)

            @pl.when(g + _NBUF - 1 < _NCH)
            def _f():
                fire(g + _NBUF - 1, b3)

            drain(g, b)
            put(g, b)
        return _

    lax.fori_loop(0, _NCH // _NBUF, step, 0)
    wait_put(_NCH - 1, (_NCH - 1) % _NBUF)


_sc_call = pl.kernel(
    _sc_body,
    out_type=jax.ShapeDtypeStruct((_N, _PAD_D), jnp.float32),
    mesh=plsc.VectorSubcoreMesh(core_axis_name="c", subcore_axis_name="s"),
    compiler_params=pltpu.CompilerParams(use_tc_tiling_on_sc=False),
    scratch_types=[
        pltpu.VMEM((_ROWS_W,), jnp.int32),
        pltpu.VMEM((_ROWS_W,), jnp.int32),
        pltpu.VMEM((_BLK,), jnp.int32),
        pltpu.VMEM((_BLK,), jnp.int32),
        pltpu.VMEM((_BLK,), jnp.int32),
        pltpu.VMEM((_NBUF, _CG, _TOK_D), jnp.float32),
        pltpu.VMEM((_NBUF, _CG, _SMALL_D), jnp.float32),
        pltpu.SemaphoreType.DMA,
        pltpu.SemaphoreType.DMA,
        pltpu.SemaphoreType.DMA((_NBUF,)),
        pltpu.SemaphoreType.DMA((_NBUF,)),
        pltpu.SemaphoreType.DMA((_NBUF,)),
    ],
)


def _tout_body(src_ref, dst_ref):
    dst_ref[0] = src_ref[:, : _OUT_D].T


_tout = pl.pallas_call(
    _tout_body,
    grid=(_L,),
    in_specs=[pl.BlockSpec((_B, _PAD_D), lambda l: (l, 0))],
    out_specs=pl.BlockSpec((1, _OUT_D, _B), lambda l: (l, 0, 0)),
    out_shape=jax.ShapeDtypeStruct((_L, _OUT_D, _B), jnp.float32),
)


def kernel(token, week, hour, duration, token_table, week_table, hour_table,
           duration_table):
    comb = jnp.concatenate([
        jnp.broadcast_to(week_table[:, None, None, :], (7, 24, 24, 16)),
        jnp.broadcast_to(hour_table[None, :, None, :], (7, 24, 24, 16)),
        jnp.broadcast_to(duration_table[None, None, :, :], (7, 24, 24, 16)),
    ], axis=-1).reshape(7 * 24 * 24, _SMALL_D)

    tab_lin = _detile(token_table.T).reshape(_V, _TOK_D)
    rows = _sc_call(token.T.reshape(_N), week.T.reshape(_N),
                    hour.T.reshape(_N), duration.T.reshape(_N),
                    tab_lin, comb)
    out3 = _tout(rows)
    return out3.transpose(2, 0, 1)


# speedup vs baseline: 9.9456x; 1.0290x over previous
import functools

import jax
import jax.numpy as jnp
from jax import lax
from jax.experimental import pallas as pl
from jax.experimental.pallas import tpu as pltpu
from jax.experimental.pallas import tpu_sc as plsc

_B, _L = 4096, 200
_N = _B * _L
_V = 1000000
_TOK_D = 64
_SMALL_D = 48
_OUT_D = _TOK_D + _SMALL_D
_PAD_D = 128
_NC, _NS = 2, 16
_NW = _NC * _NS
_LH = _L // 2
_NH = _N // 2
_ROWS_W = _NH // _NW
_CG = 128
_NCH = _ROWS_W // _CG
_NBUF = 4
_BLK = 3200
_NBLK = _ROWS_W // _BLK
_VB = 8192


def _detile_body(src_ref, dst_ref):
    x = src_ref[...]
    xp = jnp.pad(x, ((0, 128 - _TOK_D), (0, 0)))
    t = xp.T
    z = t.reshape(_VB // 2, 2, 128)
    dst_ref[:, 0:_TOK_D] = z[:, 0, 0:_TOK_D]
    dst_ref[:, _TOK_D:128] = z[:, 1, 0:_TOK_D]


_detile = pl.pallas_call(
    _detile_body,
    grid=((_V + _VB - 1) // _VB,),
    in_specs=[pl.BlockSpec((_TOK_D, _VB), lambda j: (0, j))],
    out_specs=pl.BlockSpec((_VB // 2, 128), lambda j: (j, 0)),
    out_shape=jax.ShapeDtypeStruct((_V // 2, 128), jnp.float32),
)


def _sc_body(half_base,
             tok_hbm, wk_hbm, hr_hbm, du_hbm, tok_tab, comb_tab, out_hbm,
             tok_idx, cidx, wbuf, hbuf, dbuf, tok_rows, small_rows,
             sem_tok, sem_idx, sem_t, sem_s, sem_w):
    wid = lax.axis_index("c") * _NS + lax.axis_index("s")
    lbase = pl.multiple_of(wid * _ROWS_W, _ROWS_W)
    rbase = pl.multiple_of(half_base + wid * _ROWS_W, _ROWS_W)

    tok_cp = pltpu.async_copy(tok_hbm.at[pl.ds(rbase, _ROWS_W)], tok_idx,
                              sem_tok)
    for blk in range(_NBLK):
        off = rbase + blk * _BLK
        cw = pltpu.async_copy(wk_hbm.at[pl.ds(off, _BLK)], wbuf, sem_idx)
        ch = pltpu.async_copy(hr_hbm.at[pl.ds(off, _BLK)], hbuf, sem_idx)
        cd = pltpu.async_copy(du_hbm.at[pl.ds(off, _BLK)], dbuf, sem_idx)
        cw.wait()
        ch.wait()
        cd.wait()

        def fuse(j, _):
            s = pl.ds(pl.multiple_of(j * 16, 16), 16)
            w = wbuf[s]
            h = hbuf[s]
            d = dbuf[s]
            so = pl.ds(pl.multiple_of(blk * _BLK + j * 16, 16), 16)
            cidx[so] = (w * 24 + h) * 24 + d
            return _

        lax.fori_loop(0, _BLK // 16, fuse, 0)
    tok_cp.wait()

    def fire(g, slot):
        s = pl.ds(pl.multiple_of(g * _CG, _CG), _CG)
        pltpu.async_copy(tok_tab.at[tok_idx.at[s]], tok_rows.at[slot],
                         sem_t.at[slot])
        pltpu.async_copy(comb_tab.at[cidx.at[s]], small_rows.at[slot],
                         sem_s.at[slot])

    def drain(g, slot):
        s = pl.ds(pl.multiple_of(g * _CG, _CG), _CG)
        pltpu.make_async_copy(tok_tab.at[tok_idx.at[s]], tok_rows.at[slot],
                              sem_t.at[slot]).wait()
        pltpu.make_async_copy(comb_tab.at[cidx.at[s]], small_rows.at[slot],
                              sem_s.at[slot]).wait()

    def put(g, slot):
        r = pl.ds(pl.multiple_of(lbase + g * _CG, _CG), _CG)
        pltpu.async_copy(tok_rows.at[slot],
                         out_hbm.at[r, pl.ds(0, _TOK_D)], sem_w.at[slot])
        pltpu.async_copy(small_rows.at[slot],
                         out_hbm.at[r, pl.ds(_TOK_D, _SMALL_D)],
                         sem_w.at[slot])

    def wait_put(g, slot):
        r = pl.ds(pl.multiple_of(lbase + g * _CG, _CG), _CG)
        pltpu.make_async_copy(tok_rows.at[slot],
                              out_hbm.at[r, pl.ds(0, _TOK_D)],
                              sem_w.at[slot]).wait()
        pltpu.make_async_copy(small_rows.at[slot],
                              out_hbm.at[r, pl.ds(_TOK_D, _SMALL_D)],
                              sem_w.at[slot]).wait()

    for g in range(_NBUF - 1):
        fire(g, g)

    def step(i, _):
        for b in range(_NBUF):
            g = i * _NBUF + b
            b3 = (b + _NBUF - 1) % _NBUF

            @pl.when(g >= 1)
            def _wp():
                wait_put(g - 1, b3)

            @pl.when(g + _NBUF - 1 < _NCH)
            def _f():
                fire(g + _NBUF - 1, b3)

            drain(g, b)
            put(g, b)
        return _

    lax.fori_loop(0, _NCH // _NBUF, step, 0)
    wait_put(_NCH - 1, (_NCH - 1) % _NBUF)


def _make_sc(half_base):
    return pl.kernel(
        functools.partial(_sc_body, half_base),
        out_type=jax.ShapeDtypeStruct((_NH, _PAD_D), jnp.float32),
        mesh=plsc.VectorSubcoreMesh(core_axis_name="c",
                                    subcore_axis_name="s"),
        compiler_params=pltpu.CompilerParams(use_tc_tiling_on_sc=False),
        scratch_types=[
            pltpu.VMEM((_ROWS_W,), jnp.int32),
            pltpu.VMEM((_ROWS_W,), jnp.int32),
            pltpu.VMEM((_BLK,), jnp.int32),
            pltpu.VMEM((_BLK,), jnp.int32),
            pltpu.VMEM((_BLK,), jnp.int32),
            pltpu.VMEM((_NBUF, _CG, _TOK_D), jnp.float32),
            pltpu.VMEM((_NBUF, _CG, _SMALL_D), jnp.float32),
            pltpu.SemaphoreType.DMA,
            pltpu.SemaphoreType.DMA,
            pltpu.SemaphoreType.DMA((_NBUF,)),
            pltpu.SemaphoreType.DMA((_NBUF,)),
            pltpu.SemaphoreType.DMA((_NBUF,)),
        ],
    )


_sc_a = _make_sc(0)
_sc_b = _make_sc(_NH)


def _tout_a_body(src_ref, dst_ref):
    dst_ref[0] = src_ref[:, : _OUT_D].T


_tout_a = pl.pallas_call(
    _tout_a_body,
    grid=(_LH,),
    in_specs=[pl.BlockSpec((_B, _PAD_D), lambda l: (l, 0))],
    out_specs=pl.BlockSpec((1, _OUT_D, _B), lambda l: (l, 0, 0)),
    out_shape=jax.ShapeDtypeStruct((_L, _OUT_D, _B), jnp.float32),
)


def _tout_b_body(src_ref, acc_ref, dst_ref):
    del acc_ref
    dst_ref[0] = src_ref[:, : _OUT_D].T


_tout_b = pl.pallas_call(
    _tout_b_body,
    grid=(_LH,),
    in_specs=[
        pl.BlockSpec((_B, _PAD_D), lambda l: (l, 0)),
        pl.BlockSpec(memory_space=pl.ANY),
    ],
    out_specs=pl.BlockSpec((1, _OUT_D, _B), lambda l: (l + _LH, 0, 0)),
    out_shape=jax.ShapeDtypeStruct((_L, _OUT_D, _B), jnp.float32),
    input_output_aliases={1: 0},
)


def kernel(token, week, hour, duration, token_table, week_table, hour_table,
           duration_table):
    comb = jnp.concatenate([
        jnp.broadcast_to(week_table[:, None, None, :], (7, 24, 24, 16)),
        jnp.broadcast_to(hour_table[None, :, None, :], (7, 24, 24, 16)),
        jnp.broadcast_to(duration_table[None, None, :, :], (7, 24, 24, 16)),
    ], axis=-1).reshape(7 * 24 * 24, _SMALL_D)

    tokf = token.T.reshape(_N)
    wkf = week.T.reshape(_N)
    hrf = hour.T.reshape(_N)
    duf = duration.T.reshape(_N)
    tab_lin = _detile(token_table.T).reshape(_V, _TOK_D)

    rows_a = _sc_a(tokf, wkf, hrf, duf, tab_lin, comb)
    rows_b = _sc_b(tokf, wkf, hrf, duf, tab_lin, comb)
    o1 = _tout_a(rows_a)
    out3 = _tout_b(rows_b, o1)
    return out3.transpose(2, 0, 1)


# speedup vs baseline: 10.9158x; 1.0976x over previous
import functools

import jax
import jax.numpy as jnp
from jax import lax
from jax.experimental import pallas as pl
from jax.experimental.pallas import tpu as pltpu
from jax.experimental.pallas import tpu_sc as plsc

_B, _L = 4096, 200
_N = _B * _L
_V = 1000000
_TOK_D = 64
_SMALL_D = 48
_OUT_D = _TOK_D + _SMALL_D
_PAD_D = 128
_NC, _NS = 2, 16
_NW = _NC * _NS
_LH = _L // 2
_NH = _N // 2
_ROWS_W = _NH // _NW
_CG = 128
_NCH = _ROWS_W // _CG
_NBUF = 4
_BLK = 3200
_NBLK = _ROWS_W // _BLK
_VB = 8192


def _detile_body(src_ref, dst_ref):
    x = src_ref[...]
    xs = jnp.concatenate([x[:, 1:], x[:, :1]], axis=1)
    x2 = jnp.concatenate([x, xs], axis=0)
    t2 = x2.T
    z = t2.reshape(_VB // 2, 2, 128)
    dst_ref[...] = z[:, 0, :]


_detile = pl.pallas_call(
    _detile_body,
    grid=((_V + _VB - 1) // _VB,),
    in_specs=[pl.BlockSpec((_TOK_D, _VB), lambda j: (0, j))],
    out_specs=pl.BlockSpec((_VB // 2, 128), lambda j: (j, 0)),
    out_shape=jax.ShapeDtypeStruct((_V // 2, 128), jnp.float32),
)


def _sc_body(half_base,
             tok_hbm, wk_hbm, hr_hbm, du_hbm, tok_tab, comb_tab, out_hbm,
             tok_idx, cidx, wbuf, hbuf, dbuf, tok_rows, small_rows,
             sem_tok, sem_idx, sem_t, sem_s, sem_w):
    wid = lax.axis_index("c") * _NS + lax.axis_index("s")
    lbase = pl.multiple_of(wid * _ROWS_W, _ROWS_W)
    rbase = pl.multiple_of(half_base + wid * _ROWS_W, _ROWS_W)

    tok_cp = pltpu.async_copy(tok_hbm.at[pl.ds(rbase, _ROWS_W)], tok_idx,
                              sem_tok)
    for blk in range(_NBLK):
        off = rbase + blk * _BLK
        cw = pltpu.async_copy(wk_hbm.at[pl.ds(off, _BLK)], wbuf, sem_idx)
        ch = pltpu.async_copy(hr_hbm.at[pl.ds(off, _BLK)], hbuf, sem_idx)
        cd = pltpu.async_copy(du_hbm.at[pl.ds(off, _BLK)], dbuf, sem_idx)
        cw.wait()
        ch.wait()
        cd.wait()

        def fuse(j, _):
            s = pl.ds(pl.multiple_of(j * 16, 16), 16)
            w = wbuf[s]
            h = hbuf[s]
            d = dbuf[s]
            so = pl.ds(pl.multiple_of(blk * _BLK + j * 16, 16), 16)
            cidx[so] = (w * 24 + h) * 24 + d
            return _

        lax.fori_loop(0, _BLK // 16, fuse, 0)
    tok_cp.wait()

    def fire(g, slot):
        s = pl.ds(pl.multiple_of(g * _CG, _CG), _CG)
        pltpu.async_copy(tok_tab.at[tok_idx.at[s]], tok_rows.at[slot],
                         sem_t.at[slot])
        pltpu.async_copy(comb_tab.at[cidx.at[s]], small_rows.at[slot],
                         sem_s.at[slot])

    def drain(g, slot):
        s = pl.ds(pl.multiple_of(g * _CG, _CG), _CG)
        pltpu.make_async_copy(tok_tab.at[tok_idx.at[s]], tok_rows.at[slot],
                              sem_t.at[slot]).wait()
        pltpu.make_async_copy(comb_tab.at[cidx.at[s]], small_rows.at[slot],
                              sem_s.at[slot]).wait()

    def put(g, slot):
        r = pl.ds(pl.multiple_of(lbase + g * _CG, _CG), _CG)
        pltpu.async_copy(tok_rows.at[slot],
                         out_hbm.at[r, pl.ds(0, _TOK_D)], sem_w.at[slot])
        pltpu.async_copy(small_rows.at[slot],
                         out_hbm.at[r, pl.ds(_TOK_D, _SMALL_D)],
                         sem_w.at[slot])

    def wait_put(g, slot):
        r = pl.ds(pl.multiple_of(lbase + g * _CG, _CG), _CG)
        pltpu.make_async_copy(tok_rows.at[slot],
                              out_hbm.at[r, pl.ds(0, _TOK_D)],
                              sem_w.at[slot]).wait()
        pltpu.make_async_copy(small_rows.at[slot],
                              out_hbm.at[r, pl.ds(_TOK_D, _SMALL_D)],
                              sem_w.at[slot]).wait()

    for g in range(_NBUF - 1):
        fire(g, g)

    def step(i, _):
        for b in range(_NBUF):
            g = i * _NBUF + b
            b3 = (b + _NBUF - 1) % _NBUF

            @pl.when(g >= 1)
            def _wp():
                wait_put(g - 1, b3)

            @pl.when(g + _NBUF - 1 < _NCH)
            def _f():
                fire(g + _NBUF - 1, b3)

            drain(g, b)
            put(g, b)
        return _

    lax.fori_loop(0, _NCH // _NBUF, step, 0)
    wait_put(_NCH - 1, (_NCH - 1) % _NBUF)


def _make_sc(half_base):
    return pl.kernel(
        functools.partial(_sc_body, half_base),
        out_type=jax.ShapeDtypeStruct((_NH, _PAD_D), jnp.float32),
        mesh=plsc.VectorSubcoreMesh(core_axis_name="c",
                                    subcore_axis_name="s"),
        compiler_params=pltpu.CompilerParams(use_tc_tiling_on_sc=False),
        scratch_types=[
            pltpu.VMEM((_ROWS_W,), jnp.int32),
            pltpu.VMEM((_ROWS_W,), jnp.int32),
            pltpu.VMEM((_BLK,), jnp.int32),
            pltpu.VMEM((_BLK,), jnp.int32),
            pltpu.VMEM((_BLK,), jnp.int32),
            pltpu.VMEM((_NBUF, _CG, _TOK_D), jnp.float32),
            pltpu.VMEM((_NBUF, _CG, _SMALL_D), jnp.float32),
            pltpu.SemaphoreType.DMA,
            pltpu.SemaphoreType.DMA,
            pltpu.SemaphoreType.DMA((_NBUF,)),
            pltpu.SemaphoreType.DMA((_NBUF,)),
            pltpu.SemaphoreType.DMA((_NBUF,)),
        ],
    )


_sc_a = _make_sc(0)
_sc_b = _make_sc(_NH)


def _tout_a_body(src_ref, dst_ref):
    dst_ref[0] = src_ref[:, : _OUT_D].T


_tout_a = pl.pallas_call(
    _tout_a_body,
    grid=(_LH,),
    in_specs=[pl.BlockSpec((_B, _PAD_D), lambda l: (l, 0))],
    out_specs=pl.BlockSpec((1, _OUT_D, _B), lambda l: (l, 0, 0)),
    out_shape=jax.ShapeDtypeStruct((_L, _OUT_D, _B), jnp.float32),
)


def _tout_b_body(src_ref, acc_ref, dst_ref):
    del acc_ref
    dst_ref[0] = src_ref[:, : _OUT_D].T


_tout_b = pl.pallas_call(
    _tout_b_body,
    grid=(_LH,),
    in_specs=[
        pl.BlockSpec((_B, _PAD_D), lambda l: (l, 0)),
        pl.BlockSpec(memory_space=pl.ANY),
    ],
    out_specs=pl.BlockSpec((1, _OUT_D, _B), lambda l: (l + _LH, 0, 0)),
    out_shape=jax.ShapeDtypeStruct((_L, _OUT_D, _B), jnp.float32),
    input_output_aliases={1: 0},
)


def kernel(token, week, hour, duration, token_table, week_table, hour_table,
           duration_table):
    comb = jnp.concatenate([
        jnp.broadcast_to(week_table[:, None, None, :], (7, 24, 24, 16)),
        jnp.broadcast_to(hour_table[None, :, None, :], (7, 24, 24, 16)),
        jnp.broadcast_to(duration_table[None, None, :, :], (7, 24, 24, 16)),
    ], axis=-1).reshape(7 * 24 * 24, _SMALL_D)

    tokf = token.T.reshape(_N)
    wkf = week.T.reshape(_N)
    hrf = hour.T.reshape(_N)
    duf = duration.T.reshape(_N)
    tab_lin = _detile(token_table.T).reshape(_V, _TOK_D)

    rows_a = _sc_a(tokf, wkf, hrf, duf, tab_lin, comb)
    rows_b = _sc_b(tokf, wkf, hrf, duf, tab_lin, comb)
    o1 = _tout_a(rows_a)
    out3 = _tout_b(rows_b, o1)
    return out3.transpose(2, 0, 1)


# speedup vs baseline: 11.1365x; 1.0202x over previous
import functools

import jax
import jax.numpy as jnp
from jax import lax
from jax.experimental import pallas as pl
from jax.experimental.pallas import tpu as pltpu
from jax.experimental.pallas import tpu_sc as plsc

_B, _L = 4096, 200
_N = _B * _L
_V = 1000000
_TOK_D = 64
_SMALL_D = 48
_OUT_D = _TOK_D + _SMALL_D
_PAD_D = 128
_NC, _NS = 2, 16
_NW = _NC * _NS
_PARTS = 4
_LP = _L // _PARTS
_NP = _N // _PARTS
_ROWS_W = _NP // _NW
_CG = 128
_NCH = _ROWS_W // _CG
_NBUF = 5
_BLK = 3200
_NBLK = _ROWS_W // _BLK
_VB = 8192


def _detile_body(src_ref, dst_ref):
    x = src_ref[...]
    xs = jnp.concatenate([x[:, 1:], x[:, :1]], axis=1)
    x2 = jnp.concatenate([x, xs], axis=0)
    t2 = x2.T
    z = t2.reshape(_VB // 2, 2, 128)
    dst_ref[...] = z[:, 0, :]


_detile = pl.pallas_call(
    _detile_body,
    grid=((_V + _VB - 1) // _VB,),
    in_specs=[pl.BlockSpec((_TOK_D, _VB), lambda j: (0, j))],
    out_specs=pl.BlockSpec((_VB // 2, 128), lambda j: (j, 0)),
    out_shape=jax.ShapeDtypeStruct((_V // 2, 128), jnp.float32),
)


def _sc_body(half_base,
             tok_hbm, wk_hbm, hr_hbm, du_hbm, tok_tab, comb_tab, out_hbm,
             tok_idx, cidx, wbuf, hbuf, dbuf, tok_rows, small_rows,
             sem_tok, sem_idx, sem_t, sem_s, sem_w):
    wid = lax.axis_index("c") * _NS + lax.axis_index("s")
    lbase = pl.multiple_of(wid * _ROWS_W, _ROWS_W)
    rbase = pl.multiple_of(half_base + wid * _ROWS_W, _ROWS_W)

    tok_cp = pltpu.async_copy(tok_hbm.at[pl.ds(rbase, _ROWS_W)], tok_idx,
                              sem_tok)
    for blk in range(_NBLK):
        off = rbase + blk * _BLK
        cw = pltpu.async_copy(wk_hbm.at[pl.ds(off, _BLK)], wbuf, sem_idx)
        ch = pltpu.async_copy(hr_hbm.at[pl.ds(off, _BLK)], hbuf, sem_idx)
        cd = pltpu.async_copy(du_hbm.at[pl.ds(off, _BLK)], dbuf, sem_idx)
        cw.wait()
        ch.wait()
        cd.wait()

        def fuse(j, _):
            s = pl.ds(pl.multiple_of(j * 16, 16), 16)
            w = wbuf[s]
            h = hbuf[s]
            d = dbuf[s]
            so = pl.ds(pl.multiple_of(blk * _BLK + j * 16, 16), 16)
            cidx[so] = (w * 24 + h) * 24 + d
            return _

        lax.fori_loop(0, _BLK // 16, fuse, 0)
    tok_cp.wait()

    def fire(g, slot):
        s = pl.ds(pl.multiple_of(g * _CG, _CG), _CG)
        pltpu.async_copy(tok_tab.at[tok_idx.at[s]], tok_rows.at[slot],
                         sem_t.at[slot])
        pltpu.async_copy(comb_tab.at[cidx.at[s]], small_rows.at[slot],
                         sem_s.at[slot])

    def drain(g, slot):
        s = pl.ds(pl.multiple_of(g * _CG, _CG), _CG)
        pltpu.make_async_copy(tok_tab.at[tok_idx.at[s]], tok_rows.at[slot],
                              sem_t.at[slot]).wait()
        pltpu.make_async_copy(comb_tab.at[cidx.at[s]], small_rows.at[slot],
                              sem_s.at[slot]).wait()

    def put(g, slot):
        r = pl.ds(pl.multiple_of(lbase + g * _CG, _CG), _CG)
        pltpu.async_copy(tok_rows.at[slot],
                         out_hbm.at[r, pl.ds(0, _TOK_D)], sem_w.at[slot])
        pltpu.async_copy(small_rows.at[slot],
                         out_hbm.at[r, pl.ds(_TOK_D, _SMALL_D)],
                         sem_w.at[slot])

    def wait_put(g, slot):
        r = pl.ds(pl.multiple_of(lbase + g * _CG, _CG), _CG)
        pltpu.make_async_copy(tok_rows.at[slot],
                              out_hbm.at[r, pl.ds(0, _TOK_D)],
                              sem_w.at[slot]).wait()
        pltpu.make_async_copy(small_rows.at[slot],
                              out_hbm.at[r, pl.ds(_TOK_D, _SMALL_D)],
                              sem_w.at[slot]).wait()

    for g in range(_NBUF - 1):
        fire(g, g)

    def step(i, _):
        for b in range(_NBUF):
            g = i * _NBUF + b
            b3 = (b + _NBUF - 1) % _NBUF

            @pl.when(g >= 1)
            def _wp():
                wait_put(g - 1, b3)

            @pl.when(g + _NBUF - 1 < _NCH)
            def _f():
                fire(g + _NBUF - 1, b3)

            drain(g, b)
            put(g, b)
        return _

    lax.fori_loop(0, _NCH // _NBUF, step, 0)
    wait_put(_NCH - 1, (_NCH - 1) % _NBUF)


def _make_sc(half_base):
    return pl.kernel(
        functools.partial(_sc_body, half_base),
        out_type=jax.ShapeDtypeStruct((_NP, _PAD_D), jnp.float32),
        mesh=plsc.VectorSubcoreMesh(core_axis_name="c",
                                    subcore_axis_name="s"),
        compiler_params=pltpu.CompilerParams(use_tc_tiling_on_sc=False),
        scratch_types=[
            pltpu.VMEM((_ROWS_W,), jnp.int32),
            pltpu.VMEM((_ROWS_W,), jnp.int32),
            pltpu.VMEM((_BLK,), jnp.int32),
            pltpu.VMEM((_BLK,), jnp.int32),
            pltpu.VMEM((_BLK,), jnp.int32),
            pltpu.VMEM((_NBUF, _CG, _TOK_D), jnp.float32),
            pltpu.VMEM((_NBUF, _CG, _SMALL_D), jnp.float32),
            pltpu.SemaphoreType.DMA,
            pltpu.SemaphoreType.DMA,
            pltpu.SemaphoreType.DMA((_NBUF,)),
            pltpu.SemaphoreType.DMA((_NBUF,)),
            pltpu.SemaphoreType.DMA((_NBUF,)),
        ],
    )


_sc_parts = [_make_sc(k * _NP) for k in range(_PARTS)]


def _tout0_body(src_ref, dst_ref):
    dst_ref[0] = src_ref[:, : _OUT_D].T


def _toutk_body(src_ref, acc_ref, dst_ref):
    del acc_ref
    dst_ref[0] = src_ref[:, : _OUT_D].T


def _make_tout(k):
    if k == 0:
        return pl.pallas_call(
            _tout0_body,
            grid=(_LP,),
            in_specs=[pl.BlockSpec((_B, _PAD_D), lambda l: (l, 0))],
            out_specs=pl.BlockSpec((1, _OUT_D, _B), lambda l: (l, 0, 0)),
            out_shape=jax.ShapeDtypeStruct((_L, _OUT_D, _B), jnp.float32),
        )
    return pl.pallas_call(
        _toutk_body,
        grid=(_LP,),
        in_specs=[
            pl.BlockSpec((_B, _PAD_D), lambda l: (l, 0)),
            pl.BlockSpec(memory_space=pl.ANY),
        ],
        out_specs=pl.BlockSpec((1, _OUT_D, _B),
                               lambda l, _k=k: (l + _k * _LP, 0, 0)),
        out_shape=jax.ShapeDtypeStruct((_L, _OUT_D, _B), jnp.float32),
        input_output_aliases={1: 0},
    )


_tout_parts = [_make_tout(k) for k in range(_PARTS)]


def kernel(token, week, hour, duration, token_table, week_table, hour_table,
           duration_table):
    comb = jnp.concatenate([
        jnp.broadcast_to(week_table[:, None, None, :], (7, 24, 24, 16)),
        jnp.broadcast_to(hour_table[None, :, None, :], (7, 24, 24, 16)),
        jnp.broadcast_to(duration_table[None, None, :, :], (7, 24, 24, 16)),
    ], axis=-1).reshape(7 * 24 * 24, _SMALL_D)

    tokf = token.T.reshape(_N)
    wkf = week.T.reshape(_N)
    hrf = hour.T.reshape(_N)
    duf = duration.T.reshape(_N)
    tab_lin = _detile(token_table.T).reshape(_V, _TOK_D)

    rows = [sc(tokf, wkf, hrf, duf, tab_lin, comb) for sc in _sc_parts]
    o = _tout_parts[0](rows[0])
    for k in range(1, _PARTS):
        o = _tout_parts[k](rows[k], o)
    return o.transpose(2, 0, 1)


# speedup vs baseline: 11.1479x; 1.0010x over previous
import functools

import jax
import jax.numpy as jnp
from jax import lax
from jax.experimental import pallas as pl
from jax.experimental.pallas import tpu as pltpu
from jax.experimental.pallas import tpu_sc as plsc

_B, _L = 4096, 200
_N = _B * _L
_V = 1000000
_TOK_D = 64
_SMALL_D = 48
_OUT_D = _TOK_D + _SMALL_D
_PAD_D = 128
_NC, _NS = 2, 16
_NW = _NC * _NS
_PARTS = 4
_LP = _L // _PARTS
_NP = _N // _PARTS
_ROWS_W = _NP // _NW
_CG = 128
_NCH = _ROWS_W // _CG
_NBUF = 5
_BLK = 3200
_NBLK = _ROWS_W // _BLK
_VB = 8192


def _detile_body(src_ref, dst_ref):
    x = src_ref[...]
    xs = jnp.concatenate([x[:, 1:], x[:, :1]], axis=1)
    x2 = jnp.concatenate([x, xs], axis=0)
    t2 = x2.T
    z = t2.reshape(_VB // 2, 2, 128)
    dst_ref[...] = z[:, 0, :]


_detile = pl.pallas_call(
    _detile_body,
    grid=((_V + _VB - 1) // _VB,),
    in_specs=[pl.BlockSpec((_TOK_D, _VB), lambda j: (0, j))],
    out_specs=pl.BlockSpec((_VB // 2, 128), lambda j: (j, 0)),
    out_shape=jax.ShapeDtypeStruct((_V // 2, 128), jnp.float32),
)


def _sc_body(half_base,
             tok_hbm, wk_hbm, hr_hbm, du_hbm, tok_tab, comb_tab, out_hbm,
             tok_idx, cidx, wbuf, hbuf, dbuf, tok_rows, small_rows,
             sem_tok, sem_idx, sem_t, sem_s, sem_w):
    wid = lax.axis_index("c") * _NS + lax.axis_index("s")
    lbase = pl.multiple_of(wid * _ROWS_W, _ROWS_W)
    rbase = pl.multiple_of(half_base + wid * _ROWS_W, _ROWS_W)

    tok_cp = pltpu.async_copy(tok_hbm.at[pl.ds(rbase, _ROWS_W)], tok_idx,
                              sem_tok)
    for blk in range(_NBLK):
        off = rbase + blk * _BLK
        cw = pltpu.async_copy(wk_hbm.at[pl.ds(off, _BLK)], wbuf, sem_idx)
        ch = pltpu.async_copy(hr_hbm.at[pl.ds(off, _BLK)], hbuf, sem_idx)
        cd = pltpu.async_copy(du_hbm.at[pl.ds(off, _BLK)], dbuf, sem_idx)
        cw.wait()
        ch.wait()
        cd.wait()

        def fuse(j, _):
            s = pl.ds(pl.multiple_of(j * 16, 16), 16)
            w = wbuf[s]
            h = hbuf[s]
            d = dbuf[s]
            so = pl.ds(pl.multiple_of(blk * _BLK + j * 16, 16), 16)
            cidx[so] = (w * 24 + h) * 24 + d
            return _

        lax.fori_loop(0, _BLK // 16, fuse, 0)
    tok_cp.wait()

    def fire(g, slot):
        s = pl.ds(pl.multiple_of(g * _CG, _CG), _CG)
        pltpu.async_copy(tok_tab.at[tok_idx.at[s]], tok_rows.at[slot],
                         sem_t.at[slot])
        pltpu.async_copy(comb_tab.at[cidx.at[s]], small_rows.at[slot],
                         sem_s.at[slot])

    def drain(g, slot):
        s = pl.ds(pl.multiple_of(g * _CG, _CG), _CG)
        pltpu.make_async_copy(tok_tab.at[tok_idx.at[s]], tok_rows.at[slot],
                              sem_t.at[slot]).wait()
        pltpu.make_async_copy(comb_tab.at[cidx.at[s]], small_rows.at[slot],
                              sem_s.at[slot]).wait()

    def put(g, slot):
        r = pl.ds(pl.multiple_of(lbase + g * _CG, _CG), _CG)
        pltpu.async_copy(tok_rows.at[slot],
                         out_hbm.at[r, pl.ds(0, _TOK_D)], sem_w.at[slot])
        pltpu.async_copy(small_rows.at[slot],
                         out_hbm.at[r, pl.ds(_TOK_D, _SMALL_D)],
                         sem_w.at[slot])

    def wait_put(g, slot):
        r = pl.ds(pl.multiple_of(lbase + g * _CG, _CG), _CG)
        pltpu.make_async_copy(tok_rows.at[slot],
                              out_hbm.at[r, pl.ds(0, _TOK_D)],
                              sem_w.at[slot]).wait()
        pltpu.make_async_copy(small_rows.at[slot],
                              out_hbm.at[r, pl.ds(_TOK_D, _SMALL_D)],
                              sem_w.at[slot]).wait()

    for g in range(_NBUF - 1):
        fire(g, g)

    def step(i, _):
        for b in range(_NBUF):
            g = i * _NBUF + b
            b3 = (b + _NBUF - 1) % _NBUF

            @pl.when(g >= 1)
            def _wp():
                wait_put(g - 1, b3)

            @pl.when(g + _NBUF - 1 < _NCH)
            def _f():
                fire(g + _NBUF - 1, b3)

            drain(g, b)
            put(g, b)
        return _

    lax.fori_loop(0, _NCH // _NBUF, step, 0)
    wait_put(_NCH - 1, (_NCH - 1) % _NBUF)


def _make_sc(half_base):
    return pl.kernel(
        functools.partial(_sc_body, half_base),
        out_type=jax.ShapeDtypeStruct((_NP, _PAD_D), jnp.float32),
        mesh=plsc.VectorSubcoreMesh(core_axis_name="c",
                                    subcore_axis_name="s",
                                    num_cores=_NC, num_subcores=_NS),
        compiler_params=pltpu.CompilerParams(use_tc_tiling_on_sc=False),
        scratch_types=[
            pltpu.VMEM((_ROWS_W,), jnp.int32),
            pltpu.VMEM((_ROWS_W,), jnp.int32),
            pltpu.VMEM((_BLK,), jnp.int32),
            pltpu.VMEM((_BLK,), jnp.int32),
            pltpu.VMEM((_BLK,), jnp.int32),
            pltpu.VMEM((_NBUF, _CG, _TOK_D), jnp.float32),
            pltpu.VMEM((_NBUF, _CG, _SMALL_D), jnp.float32),
            pltpu.SemaphoreType.DMA,
            pltpu.SemaphoreType.DMA,
            pltpu.SemaphoreType.DMA((_NBUF,)),
            pltpu.SemaphoreType.DMA((_NBUF,)),
            pltpu.SemaphoreType.DMA((_NBUF,)),
        ],
    )


@functools.lru_cache(maxsize=1)
def _sc_parts():
    return [_make_sc(k * _NP) for k in range(_PARTS)]


def _tout0_body(src_ref, dst_ref):
    dst_ref[0] = src_ref[:, : _OUT_D].T


def _toutk_body(src_ref, acc_ref, dst_ref):
    del acc_ref
    dst_ref[0] = src_ref[:, : _OUT_D].T


def _make_tout(k):
    if k == 0:
        return pl.pallas_call(
            _tout0_body,
            grid=(_LP,),
            in_specs=[pl.BlockSpec((_B, _PAD_D), lambda l: (l, 0))],
            out_specs=pl.BlockSpec((1, _OUT_D, _B), lambda l: (l, 0, 0)),
            out_shape=jax.ShapeDtypeStruct((_L, _OUT_D, _B), jnp.float32),
        )
    return pl.pallas_call(
        _toutk_body,
        grid=(_LP,),
        in_specs=[
            pl.BlockSpec((_B, _PAD_D), lambda l: (l, 0)),
            pl.BlockSpec(memory_space=pl.ANY),
        ],
        out_specs=pl.BlockSpec((1, _OUT_D, _B),
                               lambda l, _k=k: (l + _k * _LP, 0, 0)),
        out_shape=jax.ShapeDtypeStruct((_L, _OUT_D, _B), jnp.float32),
        input_output_aliases={1: 0},
    )


_tout_parts = [_make_tout(k) for k in range(_PARTS)]


def kernel(token, week, hour, duration, token_table, week_table, hour_table,
           duration_table):
    comb = jnp.concatenate([
        jnp.broadcast_to(week_table[:, None, None, :], (7, 24, 24, 16)),
        jnp.broadcast_to(hour_table[None, :, None, :], (7, 24, 24, 16)),
        jnp.broadcast_to(duration_table[None, None, :, :], (7, 24, 24, 16)),
    ], axis=-1).reshape(7 * 24 * 24, _SMALL_D)

    tokf = token.T.reshape(_N)
    wkf = week.T.reshape(_N)
    hrf = hour.T.reshape(_N)
    duf = duration.T.reshape(_N)
    tab_lin = _detile(token_table.T).reshape(_V, _TOK_D)

    rows = [sc(tokf, wkf, hrf, duf, tab_lin, comb) for sc in _sc_parts()]
    o = _tout_parts[0](rows[0])
    for k in range(1, _PARTS):
        o = _tout_parts[k](rows[k], o)
    return o.transpose(2, 0, 1)
